# baseline (speedup 1.0000x reference)
import functools

import jax
import jax.numpy as jnp
from jax import lax
from jax.experimental import pallas as pl
from jax.experimental.pallas import tpu as pltpu
from jax.experimental.pallas import tpu_sc as plsc

NP = 10240
CP = 48
DW = 16
NW = 32
NS = 16
CH = 128
RT = NP // NS
UR = 80
BLK = 640
ALPHA = 0.1
K = 10

_MESH = plsc.VectorSubcoreMesh(core_axis_name="c", subcore_axis_name="s")
_SC_PARAMS = pltpu.CompilerParams(use_tc_tiling_on_sc=False)


def _zero_fill(ref, rows, width):
    def body(i, _):
        for j in range(width // 16):
            ref[i, pl.ds(j * 16, 16)] = jnp.zeros((16,), jnp.float32)
        return 0

    lax.fori_loop(0, rows, body, 0)


def _ones_fill(ref, rows, width):
    def body(i, _):
        for j in range(width // 16):
            ref[i, pl.ds(j * 16, 16)] = jnp.ones((16,), jnp.float32)
        return 0

    lax.fori_loop(0, rows, body, 0)


def _deg_body(nch, dst_hbm, degs_hbm, acc, zbuf, ones_v, dst_v):
    c = lax.axis_index("c")
    s = lax.axis_index("s")
    wid = c * NS + s
    _zero_fill(zbuf, RT, DW)
    pltpu.sync_copy(zbuf, acc.at[pl.ds(s * RT, RT)])
    plsc.subcore_barrier()
    _ones_fill(ones_v, CH, DW)
    pltpu.sync_copy(dst_hbm.at[wid], dst_v)

    def body(j, _):
        pltpu.sync_copy(ones_v, acc.at[dst_v.at[j]], add=True)
        return 0

    lax.fori_loop(0, nch, body, 0)
    plsc.subcore_barrier()
    pltpu.sync_copy(acc.at[pl.ds(s * RT, RT)],
                    degs_hbm.at[pl.ds(c * NP + s * RT, RT)])


def _edge_phase(nch, hhs, src_hbm, dst_hbm, accs_hbm,
                acc, src_v, dst_v, buf0, buf1, sem0, sem1, c, s):
    wid = c * NS + s
    pltpu.sync_copy(src_hbm.at[wid], src_v)
    pltpu.sync_copy(dst_hbm.at[wid], dst_v)
    plsc.subcore_barrier()

    pltpu.async_copy(hhs.at[src_v.at[0]], buf0, sem0)

    def body(j2, _):
        base = j2 * 2
        pltpu.async_copy(hhs.at[src_v.at[base + 1]], buf1, sem1)
        pltpu.make_async_copy(hhs.at[src_v.at[base]], buf0, sem0).wait()
        pltpu.sync_copy(buf0, acc.at[dst_v.at[base]], add=True)
        pltpu.async_copy(hhs.at[src_v.at[base + 2]], buf0, sem0)
        pltpu.make_async_copy(hhs.at[src_v.at[base + 1]], buf1, sem1).wait()
        pltpu.sync_copy(buf1, acc.at[dst_v.at[base + 1]], add=True)
        return 0

    lax.fori_loop(0, nch // 2, body, 0)
    pltpu.make_async_copy(hhs.at[src_v.at[nch]], buf0, sem0).wait()
    plsc.subcore_barrier()
    pltpu.sync_copy(acc.at[pl.ds(s * RT, RT)],
                    accs_hbm.at[pl.ds(c * NP + s * RT, RT)])


def _round0_body(nch, hh_hbm, src_hbm, dst_hbm, accs_hbm,
                 acc, zbuf, src_v, dst_v, buf0, buf1, sem0, sem1):
    c = lax.axis_index("c")
    s = lax.axis_index("s")

    @pl.when(c == 0)
    def _():
        pltpu.sync_copy(hh_hbm.at[pl.ds(s * RT, RT)], acc.at[pl.ds(s * RT, RT)])

    @pl.when(c != 0)
    def _():
        _zero_fill(zbuf, RT, CP)
        pltpu.sync_copy(zbuf, acc.at[pl.ds(s * RT, RT)])

    _edge_phase(nch, hh_hbm, src_hbm, dst_hbm, accs_hbm,
                acc, src_v, dst_v, buf0, buf1, sem0, sem1, c, s)


def _roundn_body(nch, accp_hbm, qz_hbm, src_hbm, dst_hbm, accs_hbm, hhc_hbm,
                 acc, src_v, dst_v, buf0, buf1,
                 a0u, a1u, qzu, sem0, sem1):
    c = lax.axis_index("c")
    s = lax.axis_index("s")
    for u in range(RT // UR):
        base = s * RT + u * UR
        pltpu.sync_copy(accp_hbm.at[pl.ds(base, UR)], a0u)
        pltpu.sync_copy(accp_hbm.at[pl.ds(NP + base, UR)], a1u)
        pltpu.sync_copy(qz_hbm.at[pl.ds(base, UR)], qzu)

        def ubody(i4, _):
            for r in range(4):
                i = i4 * 4 + r
                for j in range(CP // 16):
                    sl = pl.ds(j * 16, 16)
                    a0u[i, sl] = (qzu[i, sl] * (a0u[i, sl] + a1u[i, sl])
                                  + qzu[i, pl.ds(CP + j * 16, 16)])
            return 0

        lax.fori_loop(0, UR // 4, ubody, 0)
        pltpu.sync_copy(a0u, hhc_hbm.at[pl.ds(c * NP + base, UR)])

        @pl.when(c == 0)
        def _():
            pltpu.sync_copy(a0u, acc.at[pl.ds(base, UR)])

    @pl.when(c != 0)
    def _():
        _zero_fill(a0u, UR, CP)
        for u in range(RT // UR):
            pltpu.sync_copy(a0u, acc.at[pl.ds(s * RT + u * UR, UR)])

    _edge_phase(nch, hhc_hbm, src_hbm, dst_hbm, accs_hbm,
                acc, src_v, dst_v, buf0, buf1, sem0, sem1, c, s)


def _prep_body(x_ref, w1_ref, b1_ref, w2_ref, b2_ref, dg0_ref, dg1_ref,
               h0_ref, hh0_ref, dinv_ref, qz_ref):
    i = pl.program_id(0)
    h = jnp.maximum(
        jnp.dot(x_ref[...], w1_ref[...], preferred_element_type=jnp.float32)
        + b1_ref[...], 0.0)
    h = jnp.dot(h, w2_ref[...], preferred_element_type=jnp.float32) + b2_ref[...]
    rows = i * BLK + lax.broadcasted_iota(jnp.int32, (BLK, 1), 0)
    h = jnp.where(rows < 10000, h, 0.0)
    deg = 1.0 + dg0_ref[:, 0:1] + dg1_ref[:, 0:1]
    dinv = lax.rsqrt(deg)
    h0_ref[...] = h
    hh0_ref[...] = h * dinv
    dinv_ref[...] = dinv
    qz_ref[:, :CP] = jnp.broadcast_to((1.0 - ALPHA) * dinv * dinv, (BLK, CP))
    qz_ref[:, CP:] = ALPHA * dinv * h


def _final_body(a0_ref, a1_ref, h0_ref, dinv_ref, out_ref):
    dinv = dinv_ref[...]
    hn = (1.0 - ALPHA) * dinv * (a0_ref[...] + a1_ref[...]) \
        + ALPHA * h0_ref[...]
    l = hn[:, :40]
    m = jnp.max(l, axis=1, keepdims=True)
    e = jnp.exp(l - m)
    out_ref[...] = l - m - jnp.log(jnp.sum(e, axis=1, keepdims=True))


def kernel(x, edge_index, W1, b1, W2, b2):
    n, d = x.shape
    e = edge_index.shape[1]
    hdim = W1.shape[1]
    c0 = W2.shape[1]
    ew = e // NW
    nch = -(-ew // CH)
    nch += nch % 2
    ewp = nch * CH

    x_pad = jnp.zeros((NP, d), x.dtype).at[:n].set(x)
    W2p = jnp.zeros((hdim, CP), W2.dtype).at[:, :c0].set(W2)
    b2p = jnp.zeros((CP,), b2.dtype).at[:c0].set(b2)

    pad = ewp - ew
    pad_s = ewp + 2 * CH - ew
    pad_src = 10000 + (jnp.arange(pad_s, dtype=jnp.int32) % 64)
    pad_dst = 10016 + (jnp.arange(pad, dtype=jnp.int32) % 128)
    src3 = jnp.concatenate(
        [edge_index[0].reshape(NW, ew),
         jnp.broadcast_to(pad_src, (NW, pad_s))], axis=1).reshape(NW, nch + 2, CH)
    dst3 = jnp.concatenate(
        [edge_index[1].reshape(NW, ew),
         jnp.broadcast_to(pad_dst, (NW, pad))], axis=1).reshape(NW, nch, CH)

    deg_call = pl.kernel(
        functools.partial(_deg_body, nch),
        out_type=jax.ShapeDtypeStruct((2 * NP, DW), jnp.float32),
        mesh=_MESH,
        scratch_types=[
            pltpu.VMEM_SHARED((NP, DW), jnp.float32),
            pltpu.VMEM((RT, DW), jnp.float32),
            pltpu.VMEM((CH, DW), jnp.float32),
            pltpu.VMEM((nch, CH), jnp.int32),
        ],
        compiler_params=_SC_PARAMS,
    )
    degs = deg_call(dst3)

    prep_call = pl.pallas_call(
        _prep_body,
        grid=(NP // BLK,),
        in_specs=[
            pl.BlockSpec((BLK, d), lambda i: (i, 0)),
            pl.BlockSpec((d, hdim), lambda i: (0, 0)),
            pl.BlockSpec((hdim,), lambda i: (0,)),
            pl.BlockSpec((hdim, CP), lambda i: (0, 0)),
            pl.BlockSpec((CP,), lambda i: (0,)),
            pl.BlockSpec((BLK, DW), lambda i: (i, 0)),
            pl.BlockSpec((BLK, DW), lambda i: (i + NP // BLK, 0)),
        ],
        out_specs=[
            pl.BlockSpec((BLK, CP), lambda i: (i, 0)),
            pl.BlockSpec((BLK, CP), lambda i: (i, 0)),
            pl.BlockSpec((BLK, 1), lambda i: (i, 0)),
            pl.BlockSpec((BLK, 2 * CP), lambda i: (i, 0)),
        ],
        out_shape=[
            jax.ShapeDtypeStruct((NP, CP), jnp.float32),
            jax.ShapeDtypeStruct((NP, CP), jnp.float32),
            jax.ShapeDtypeStruct((NP, 1), jnp.float32),
            jax.ShapeDtypeStruct((NP, 2 * CP), jnp.float32),
        ],
    )
    h0, hh0, dinv, qz = prep_call(x_pad, W1, b1, W2p, b2p, degs, degs)

    edge_bufs = [
        pltpu.VMEM((nch + 2, CH), jnp.int32),
        pltpu.VMEM((nch, CH), jnp.int32),
        pltpu.VMEM((CH, CP), jnp.float32),
        pltpu.VMEM((CH, CP), jnp.float32),
    ]
    round0_call = pl.kernel(
        functools.partial(_round0_body, nch),
        out_type=jax.ShapeDtypeStruct((2 * NP, CP), jnp.float32),
        mesh=_MESH,
        scratch_types=[
            pltpu.VMEM_SHARED((NP, CP), jnp.float32),
            pltpu.VMEM((RT, CP), jnp.float32),
        ] + edge_bufs + [
            pltpu.SemaphoreType.DMA,
            pltpu.SemaphoreType.DMA,
        ],
        compiler_params=_SC_PARAMS,
    )
    roundn_call = pl.kernel(
        functools.partial(_roundn_body, nch),
        out_type=[
            jax.ShapeDtypeStruct((2 * NP, CP), jnp.float32),
            jax.ShapeDtypeStruct((2 * NP, CP), jnp.float32),
        ],
        mesh=_MESH,
        scratch_types=[
            pltpu.VMEM_SHARED((NP, CP), jnp.float32),
        ] + edge_bufs + [
            pltpu.VMEM((UR, CP), jnp.float32),
            pltpu.VMEM((UR, CP), jnp.float32),
            pltpu.VMEM((UR, 2 * CP), jnp.float32),
            pltpu.SemaphoreType.DMA,
            pltpu.SemaphoreType.DMA,
        ],
        compiler_params=_SC_PARAMS,
    )

    src3a = src3 + (jnp.arange(NW, dtype=jnp.int32)[:, None, None] // NS) * NP

    accs = round0_call(hh0, src3, dst3)
    for _ in range(K - 1):
        accs, _hhc = roundn_call(accs, qz, src3a, dst3)

    final_call = pl.pallas_call(
        _final_body,
        grid=(NP // BLK,),
        in_specs=[
            pl.BlockSpec((BLK, CP), lambda i: (i, 0)),
            pl.BlockSpec((BLK, CP), lambda i: (i + NP // BLK, 0)),
            pl.BlockSpec((BLK, CP), lambda i: (i, 0)),
            pl.BlockSpec((BLK, 1), lambda i: (i, 0)),
        ],
        out_specs=pl.BlockSpec((BLK, 40), lambda i: (i, 0)),
        out_shape=jax.ShapeDtypeStruct((NP, 40), jnp.float32),
    )
    out = final_call(accs, accs, h0, dinv)
    return out[:n]


# speedup vs baseline: 1.0532x; 1.0532x over previous
import functools

import jax
import jax.numpy as jnp
from jax import lax
from jax.experimental import pallas as pl
from jax.experimental.pallas import tpu as pltpu
from jax.experimental.pallas import tpu_sc as plsc

NP = 10240
CP = 48
DW = 16
NW = 32
NS = 16
CH = 128
RT = NP // NS
UR = 80
BLK = 640
ALPHA = 0.1
K = 10

_MESH = plsc.VectorSubcoreMesh(core_axis_name="c", subcore_axis_name="s")
_SC_PARAMS = pltpu.CompilerParams(use_tc_tiling_on_sc=False)


def _zero_fill(ref, rows, width):
    def body(i, _):
        for j in range(width // 16):
            ref[i, pl.ds(j * 16, 16)] = jnp.zeros((16,), jnp.float32)
        return 0

    lax.fori_loop(0, rows, body, 0)


def _ones_fill(ref, rows, width):
    def body(i, _):
        for j in range(width // 16):
            ref[i, pl.ds(j * 16, 16)] = jnp.ones((16,), jnp.float32)
        return 0

    lax.fori_loop(0, rows, body, 0)


def _deg_body(nch, dst_hbm, degs_hbm, acc, zbuf, ones_v, dst_v):
    c = lax.axis_index("c")
    s = lax.axis_index("s")
    wid = c * NS + s
    _zero_fill(zbuf, RT, DW)
    pltpu.sync_copy(zbuf, acc.at[pl.ds(s * RT, RT)])
    plsc.subcore_barrier()
    _ones_fill(ones_v, CH, DW)
    pltpu.sync_copy(dst_hbm.at[wid], dst_v)

    def body(j, _):
        pltpu.sync_copy(ones_v, acc.at[dst_v.at[j]], add=True)
        return 0

    lax.fori_loop(0, nch, body, 0)
    plsc.subcore_barrier()
    pltpu.sync_copy(acc.at[pl.ds(s * RT, RT)],
                    degs_hbm.at[pl.ds(c * NP + s * RT, RT)])


def _edge_phase(nch, hhs, src_hbm, dst_hbm, accs_hbm,
                acc, src_v, dst_v, buf0, buf1, sem0, sem1, c, s):
    wid = c * NS + s
    pltpu.sync_copy(src_hbm.at[wid], src_v)
    pltpu.sync_copy(dst_hbm.at[wid], dst_v)
    plsc.subcore_barrier()

    pltpu.async_copy(hhs.at[src_v.at[0]], buf0, sem0)

    def body(j2, _):
        base = j2 * 2
        pltpu.async_copy(hhs.at[src_v.at[base + 1]], buf1, sem1)
        pltpu.make_async_copy(hhs.at[src_v.at[base]], buf0, sem0).wait()
        pltpu.sync_copy(buf0, acc.at[dst_v.at[base]], add=True)
        pltpu.async_copy(hhs.at[src_v.at[base + 2]], buf0, sem0)
        pltpu.make_async_copy(hhs.at[src_v.at[base + 1]], buf1, sem1).wait()
        pltpu.sync_copy(buf1, acc.at[dst_v.at[base + 1]], add=True)
        return 0

    lax.fori_loop(0, nch // 2, body, 0)
    pltpu.make_async_copy(hhs.at[src_v.at[nch]], buf0, sem0).wait()
    plsc.subcore_barrier()
    pltpu.sync_copy(acc.at[pl.ds(s * RT, RT)],
                    accs_hbm.at[pl.ds(c * NP + s * RT, RT)])


def _round0_body(nch, hh_hbm, src_hbm, dst_hbm, accs_hbm,
                 acc, zbuf, src_v, dst_v, buf0, buf1, sem0, sem1):
    c = lax.axis_index("c")
    s = lax.axis_index("s")

    @pl.when(c == 0)
    def _():
        pltpu.sync_copy(hh_hbm.at[pl.ds(s * RT, RT)], acc.at[pl.ds(s * RT, RT)])

    @pl.when(c != 0)
    def _():
        _zero_fill(zbuf, RT, CP)
        pltpu.sync_copy(zbuf, acc.at[pl.ds(s * RT, RT)])

    _edge_phase(nch, hh_hbm, src_hbm, dst_hbm, accs_hbm,
                acc, src_v, dst_v, buf0, buf1, sem0, sem1, c, s)


def _roundn_body(nch, accp_hbm, qz_hbm, src_hbm, dst_hbm, accs_hbm,
                 hhs, acc, src_v, dst_v, buf0, buf1,
                 a0u, a1u, qzu, sem0, sem1):
    c = lax.axis_index("c")
    s = lax.axis_index("s")
    _upd = jax.named_scope("upd_phase")
    _upd.__enter__()
    for u in range(RT // UR):
        base = s * RT + u * UR
        pltpu.sync_copy(accp_hbm.at[pl.ds(base, UR)], a0u)
        pltpu.sync_copy(accp_hbm.at[pl.ds(NP + base, UR)], a1u)
        pltpu.sync_copy(qz_hbm.at[pl.ds(base, UR)], qzu)

        def ubody(i4, _):
            for r in range(4):
                i = i4 * 4 + r
                for j in range(CP // 16):
                    sl = pl.ds(j * 16, 16)
                    a0u[i, sl] = (qzu[i, sl] * (a0u[i, sl] + a1u[i, sl])
                                  + qzu[i, pl.ds(CP + j * 16, 16)])
            return 0

        lax.fori_loop(0, UR // 4, ubody, 0)
        pltpu.sync_copy(a0u, hhs.at[pl.ds(base, UR)])

        @pl.when(c == 0)
        def _():
            pltpu.sync_copy(a0u, acc.at[pl.ds(base, UR)])

    @pl.when(c != 0)
    def _():
        _zero_fill(a0u, UR, CP)
        for u in range(RT // UR):
            pltpu.sync_copy(a0u, acc.at[pl.ds(s * RT + u * UR, UR)])

    _upd.__exit__(None, None, None)
    with jax.named_scope("edge_phase"):
        _edge_phase(nch, hhs, src_hbm, dst_hbm, accs_hbm,
                    acc, src_v, dst_v, buf0, buf1, sem0, sem1, c, s)


def _prep_body(x_ref, w1_ref, b1_ref, w2_ref, b2_ref, dg0_ref, dg1_ref,
               h0_ref, hh0_ref, dinv_ref, qz_ref):
    i = pl.program_id(0)
    h = jnp.maximum(
        jnp.dot(x_ref[...], w1_ref[...], preferred_element_type=jnp.float32)
        + b1_ref[...], 0.0)
    h = jnp.dot(h, w2_ref[...], preferred_element_type=jnp.float32) + b2_ref[...]
    rows = i * BLK + lax.broadcasted_iota(jnp.int32, (BLK, 1), 0)
    h = jnp.where(rows < 10000, h, 0.0)
    deg = 1.0 + dg0_ref[:, 0:1] + dg1_ref[:, 0:1]
    dinv = lax.rsqrt(deg)
    h0_ref[...] = h
    hh0_ref[...] = h * dinv
    dinv_ref[...] = dinv
    qz_ref[:, :CP] = jnp.broadcast_to((1.0 - ALPHA) * dinv * dinv, (BLK, CP))
    qz_ref[:, CP:] = ALPHA * dinv * h


def _final_body(a0_ref, a1_ref, h0_ref, dinv_ref, out_ref):
    dinv = dinv_ref[...]
    hn = (1.0 - ALPHA) * dinv * (a0_ref[...] + a1_ref[...]) \
        + ALPHA * h0_ref[...]
    l = hn[:, :40]
    m = jnp.max(l, axis=1, keepdims=True)
    e = jnp.exp(l - m)
    out_ref[...] = l - m - jnp.log(jnp.sum(e, axis=1, keepdims=True))


def kernel(x, edge_index, W1, b1, W2, b2):
    n, d = x.shape
    e = edge_index.shape[1]
    hdim = W1.shape[1]
    c0 = W2.shape[1]
    ew = e // NW
    nch = -(-ew // CH)
    nch += nch % 2
    ewp = nch * CH

    x_pad = jnp.zeros((NP, d), x.dtype).at[:n].set(x)
    W2p = jnp.zeros((hdim, CP), W2.dtype).at[:, :c0].set(W2)
    b2p = jnp.zeros((CP,), b2.dtype).at[:c0].set(b2)

    pad = ewp - ew
    pad_s = ewp + 2 * CH - ew
    pad_src = 10000 + (jnp.arange(pad_s, dtype=jnp.int32) % 64)
    pad_dst = 10016 + (jnp.arange(pad, dtype=jnp.int32) % 128)
    src3 = jnp.concatenate(
        [edge_index[0].reshape(NW, ew),
         jnp.broadcast_to(pad_src, (NW, pad_s))], axis=1).reshape(NW, nch + 2, CH)
    dst3 = jnp.concatenate(
        [edge_index[1].reshape(NW, ew),
         jnp.broadcast_to(pad_dst, (NW, pad))], axis=1).reshape(NW, nch, CH)

    deg_call = pl.kernel(
        functools.partial(_deg_body, nch),
        out_type=jax.ShapeDtypeStruct((2 * NP, DW), jnp.float32),
        mesh=_MESH,
        scratch_types=[
            pltpu.VMEM_SHARED((NP, DW), jnp.float32),
            pltpu.VMEM((RT, DW), jnp.float32),
            pltpu.VMEM((CH, DW), jnp.float32),
            pltpu.VMEM((nch, CH), jnp.int32),
        ],
        compiler_params=_SC_PARAMS,
    )
    degs = deg_call(dst3)

    prep_call = pl.pallas_call(
        _prep_body,
        grid=(NP // BLK,),
        in_specs=[
            pl.BlockSpec((BLK, d), lambda i: (i, 0)),
            pl.BlockSpec((d, hdim), lambda i: (0, 0)),
            pl.BlockSpec((hdim,), lambda i: (0,)),
            pl.BlockSpec((hdim, CP), lambda i: (0, 0)),
            pl.BlockSpec((CP,), lambda i: (0,)),
            pl.BlockSpec((BLK, DW), lambda i: (i, 0)),
            pl.BlockSpec((BLK, DW), lambda i: (i + NP // BLK, 0)),
        ],
        out_specs=[
            pl.BlockSpec((BLK, CP), lambda i: (i, 0)),
            pl.BlockSpec((BLK, CP), lambda i: (i, 0)),
            pl.BlockSpec((BLK, 1), lambda i: (i, 0)),
            pl.BlockSpec((BLK, 2 * CP), lambda i: (i, 0)),
        ],
        out_shape=[
            jax.ShapeDtypeStruct((NP, CP), jnp.float32),
            jax.ShapeDtypeStruct((NP, CP), jnp.float32),
            jax.ShapeDtypeStruct((NP, 1), jnp.float32),
            jax.ShapeDtypeStruct((NP, 2 * CP), jnp.float32),
        ],
    )
    h0, hh0, dinv, qz = prep_call(x_pad, W1, b1, W2p, b2p, degs, degs)

    edge_bufs = [
        pltpu.VMEM((nch + 2, CH), jnp.int32),
        pltpu.VMEM((nch, CH), jnp.int32),
        pltpu.VMEM((CH, CP), jnp.float32),
        pltpu.VMEM((CH, CP), jnp.float32),
    ]
    round0_call = pl.kernel(
        functools.partial(_round0_body, nch),
        out_type=jax.ShapeDtypeStruct((2 * NP, CP), jnp.float32),
        mesh=_MESH,
        scratch_types=[
            pltpu.VMEM_SHARED((NP, CP), jnp.float32),
            pltpu.VMEM((RT, CP), jnp.float32),
        ] + edge_bufs + [
            pltpu.SemaphoreType.DMA,
            pltpu.SemaphoreType.DMA,
        ],
        compiler_params=_SC_PARAMS,
    )
    roundn_call = pl.kernel(
        functools.partial(_roundn_body, nch),
        out_type=jax.ShapeDtypeStruct((2 * NP, CP), jnp.float32),
        mesh=_MESH,
        scratch_types=[
            pltpu.VMEM_SHARED((NP, CP), jnp.float32),
            pltpu.VMEM_SHARED((NP, CP), jnp.float32),
        ] + edge_bufs + [
            pltpu.VMEM((UR, CP), jnp.float32),
            pltpu.VMEM((UR, CP), jnp.float32),
            pltpu.VMEM((UR, 2 * CP), jnp.float32),
            pltpu.SemaphoreType.DMA,
            pltpu.SemaphoreType.DMA,
        ],
        compiler_params=_SC_PARAMS,
    )

    accs = round0_call(hh0, src3, dst3)
    for _ in range(K - 1):
        accs = roundn_call(accs, qz, src3, dst3)

    final_call = pl.pallas_call(
        _final_body,
        grid=(NP // BLK,),
        in_specs=[
            pl.BlockSpec((BLK, CP), lambda i: (i, 0)),
            pl.BlockSpec((BLK, CP), lambda i: (i + NP // BLK, 0)),
            pl.BlockSpec((BLK, CP), lambda i: (i, 0)),
            pl.BlockSpec((BLK, 1), lambda i: (i, 0)),
        ],
        out_specs=pl.BlockSpec((BLK, 40), lambda i: (i, 0)),
        out_shape=jax.ShapeDtypeStruct((NP, 40), jnp.float32),
    )
    out = final_call(accs, accs, h0, dinv)
    return out[:n]


# speedup vs baseline: 1.1401x; 1.0825x over previous
import functools

import jax
import jax.numpy as jnp
from jax import lax
from jax.experimental import pallas as pl
from jax.experimental.pallas import tpu as pltpu
from jax.experimental.pallas import tpu_sc as plsc

NP = 10240
CP = 48
DW = 16
NW = 32
NS = 16
CH = 128
RT = NP // NS
UR = 80
BLK = 640
ALPHA = 0.1
K = 10

_MESH = plsc.VectorSubcoreMesh(core_axis_name="c", subcore_axis_name="s")
_SC_PARAMS = pltpu.CompilerParams(use_tc_tiling_on_sc=False)


def _zero_fill(ref, rows, width):
    @plsc.parallel_loop(0, rows, step=1, unroll=8)
    def _(i):
        for j in range(width // 16):
            ref[i, pl.ds(j * 16, 16)] = jnp.zeros((16,), jnp.float32)


def _ones_fill(ref, rows, width):
    @plsc.parallel_loop(0, rows, step=1, unroll=8)
    def _(i):
        for j in range(width // 16):
            ref[i, pl.ds(j * 16, 16)] = jnp.ones((16,), jnp.float32)


def _deg_body(nch, dst_hbm, degs_hbm, acc, zbuf, ones_v, dst_v):
    c = lax.axis_index("c")
    s = lax.axis_index("s")
    wid = c * NS + s
    _zero_fill(zbuf, RT, DW)
    pltpu.sync_copy(zbuf, acc.at[pl.ds(s * RT, RT)])
    plsc.subcore_barrier()
    _ones_fill(ones_v, CH, DW)
    pltpu.sync_copy(dst_hbm.at[wid], dst_v)

    def body(j, _):
        pltpu.sync_copy(ones_v, acc.at[dst_v.at[j]], add=True)
        return 0

    lax.fori_loop(0, nch, body, 0)
    plsc.subcore_barrier()
    pltpu.sync_copy(acc.at[pl.ds(s * RT, RT)],
                    degs_hbm.at[pl.ds(c * NP + s * RT, RT)])


def _edge_phase(nch, hhs, src_hbm, dst_hbm, accs_hbm,
                acc, src_v, dst_v, buf0, buf1, sem0, sem1, c, s):
    wid = c * NS + s
    pltpu.sync_copy(src_hbm.at[wid], src_v)
    pltpu.sync_copy(dst_hbm.at[wid], dst_v)
    plsc.subcore_barrier()

    pltpu.async_copy(hhs.at[src_v.at[0]], buf0, sem0)

    def body(j2, _):
        base = j2 * 2
        pltpu.async_copy(hhs.at[src_v.at[base + 1]], buf1, sem1)
        pltpu.make_async_copy(hhs.at[src_v.at[base]], buf0, sem0).wait()
        pltpu.sync_copy(buf0, acc.at[dst_v.at[base]], add=True)
        pltpu.async_copy(hhs.at[src_v.at[base + 2]], buf0, sem0)
        pltpu.make_async_copy(hhs.at[src_v.at[base + 1]], buf1, sem1).wait()
        pltpu.sync_copy(buf1, acc.at[dst_v.at[base + 1]], add=True)
        return 0

    lax.fori_loop(0, nch // 2, body, 0)
    pltpu.make_async_copy(hhs.at[src_v.at[nch]], buf0, sem0).wait()
    plsc.subcore_barrier()
    pltpu.sync_copy(acc.at[pl.ds(s * RT, RT)],
                    accs_hbm.at[pl.ds(c * NP + s * RT, RT)])


def _round0_body(nch, hh_hbm, src_hbm, dst_hbm, accs_hbm,
                 acc, zbuf, src_v, dst_v, buf0, buf1, sem0, sem1):
    c = lax.axis_index("c")
    s = lax.axis_index("s")

    @pl.when(c == 0)
    def _():
        pltpu.sync_copy(hh_hbm.at[pl.ds(s * RT, RT)], acc.at[pl.ds(s * RT, RT)])

    @pl.when(c != 0)
    def _():
        _zero_fill(zbuf, RT, CP)
        pltpu.sync_copy(zbuf, acc.at[pl.ds(s * RT, RT)])

    _edge_phase(nch, hh_hbm, src_hbm, dst_hbm, accs_hbm,
                acc, src_v, dst_v, buf0, buf1, sem0, sem1, c, s)


def _roundn_body(nch, accp_hbm, qz_hbm, src_hbm, dst_hbm, accs_hbm,
                 hhs, acc, src_v, dst_v, buf0, buf1,
                 a0u, a1u, qzu, sem0, sem1):
    c = lax.axis_index("c")
    s = lax.axis_index("s")
    _upd = jax.named_scope("upd_phase")
    _upd.__enter__()
    for u in range(RT // UR):
        base = s * RT + u * UR
        pltpu.sync_copy(accp_hbm.at[pl.ds(base, UR)], a0u)
        pltpu.sync_copy(accp_hbm.at[pl.ds(NP + base, UR)], a1u)
        pltpu.sync_copy(qz_hbm.at[pl.ds(base, UR)], qzu)

        @plsc.parallel_loop(0, UR, step=1, unroll=8)
        def _(i):
            for j in range(CP // 16):
                sl = pl.ds(j * 16, 16)
                a0u[i, sl] = (qzu[i, sl] * (a0u[i, sl] + a1u[i, sl])
                              + qzu[i, pl.ds(CP + j * 16, 16)])
        pltpu.sync_copy(a0u, hhs.at[pl.ds(base, UR)])

        @pl.when(c == 0)
        def _():
            pltpu.sync_copy(a0u, acc.at[pl.ds(base, UR)])

    @pl.when(c != 0)
    def _():
        _zero_fill(a0u, UR, CP)
        for u in range(RT // UR):
            pltpu.sync_copy(a0u, acc.at[pl.ds(s * RT + u * UR, UR)])

    _upd.__exit__(None, None, None)
    with jax.named_scope("edge_phase"):
        _edge_phase(nch, hhs, src_hbm, dst_hbm, accs_hbm,
                    acc, src_v, dst_v, buf0, buf1, sem0, sem1, c, s)


def _prep_body(x_ref, w1_ref, b1_ref, w2_ref, b2_ref, dg0_ref, dg1_ref,
               h0_ref, hh0_ref, dinv_ref, qz_ref):
    i = pl.program_id(0)
    h = jnp.maximum(
        jnp.dot(x_ref[...], w1_ref[...], preferred_element_type=jnp.float32)
        + b1_ref[...], 0.0)
    h = jnp.dot(h, w2_ref[...], preferred_element_type=jnp.float32) + b2_ref[...]
    rows = i * BLK + lax.broadcasted_iota(jnp.int32, (BLK, 1), 0)
    h = jnp.where(rows < 10000, h, 0.0)
    deg = 1.0 + dg0_ref[:, 0:1] + dg1_ref[:, 0:1]
    dinv = lax.rsqrt(deg)
    h0_ref[...] = h
    hh0_ref[...] = h * dinv
    dinv_ref[...] = dinv
    qz_ref[:, :CP] = jnp.broadcast_to((1.0 - ALPHA) * dinv * dinv, (BLK, CP))
    qz_ref[:, CP:] = ALPHA * dinv * h


def _final_body(a0_ref, a1_ref, h0_ref, dinv_ref, out_ref):
    dinv = dinv_ref[...]
    hn = (1.0 - ALPHA) * dinv * (a0_ref[...] + a1_ref[...]) \
        + ALPHA * h0_ref[...]
    l = hn[:, :40]
    m = jnp.max(l, axis=1, keepdims=True)
    e = jnp.exp(l - m)
    out_ref[...] = l - m - jnp.log(jnp.sum(e, axis=1, keepdims=True))


def kernel(x, edge_index, W1, b1, W2, b2):
    n, d = x.shape
    e = edge_index.shape[1]
    hdim = W1.shape[1]
    c0 = W2.shape[1]
    ew = e // NW
    nch = -(-ew // CH)
    nch += nch % 2
    ewp = nch * CH

    x_pad = jnp.zeros((NP, d), x.dtype).at[:n].set(x)
    W2p = jnp.zeros((hdim, CP), W2.dtype).at[:, :c0].set(W2)
    b2p = jnp.zeros((CP,), b2.dtype).at[:c0].set(b2)

    pad = ewp - ew
    pad_s = ewp + 2 * CH - ew
    pad_src = 10000 + (jnp.arange(pad_s, dtype=jnp.int32) % 64)
    pad_dst = 10016 + (jnp.arange(pad, dtype=jnp.int32) % 128)
    src3 = jnp.concatenate(
        [edge_index[0].reshape(NW, ew),
         jnp.broadcast_to(pad_src, (NW, pad_s))], axis=1).reshape(NW, nch + 2, CH)
    dst3 = jnp.concatenate(
        [edge_index[1].reshape(NW, ew),
         jnp.broadcast_to(pad_dst, (NW, pad))], axis=1).reshape(NW, nch, CH)

    deg_call = pl.kernel(
        functools.partial(_deg_body, nch),
        out_type=jax.ShapeDtypeStruct((2 * NP, DW), jnp.float32),
        mesh=_MESH,
        scratch_types=[
            pltpu.VMEM_SHARED((NP, DW), jnp.float32),
            pltpu.VMEM((RT, DW), jnp.float32),
            pltpu.VMEM((CH, DW), jnp.float32),
            pltpu.VMEM((nch, CH), jnp.int32),
        ],
        compiler_params=_SC_PARAMS,
    )
    degs = deg_call(dst3)

    prep_call = pl.pallas_call(
        _prep_body,
        grid=(NP // BLK,),
        in_specs=[
            pl.BlockSpec((BLK, d), lambda i: (i, 0)),
            pl.BlockSpec((d, hdim), lambda i: (0, 0)),
            pl.BlockSpec((hdim,), lambda i: (0,)),
            pl.BlockSpec((hdim, CP), lambda i: (0, 0)),
            pl.BlockSpec((CP,), lambda i: (0,)),
            pl.BlockSpec((BLK, DW), lambda i: (i, 0)),
            pl.BlockSpec((BLK, DW), lambda i: (i + NP // BLK, 0)),
        ],
        out_specs=[
            pl.BlockSpec((BLK, CP), lambda i: (i, 0)),
            pl.BlockSpec((BLK, CP), lambda i: (i, 0)),
            pl.BlockSpec((BLK, 1), lambda i: (i, 0)),
            pl.BlockSpec((BLK, 2 * CP), lambda i: (i, 0)),
        ],
        out_shape=[
            jax.ShapeDtypeStruct((NP, CP), jnp.float32),
            jax.ShapeDtypeStruct((NP, CP), jnp.float32),
            jax.ShapeDtypeStruct((NP, 1), jnp.float32),
            jax.ShapeDtypeStruct((NP, 2 * CP), jnp.float32),
        ],
    )
    h0, hh0, dinv, qz = prep_call(x_pad, W1, b1, W2p, b2p, degs, degs)

    edge_bufs = [
        pltpu.VMEM((nch + 2, CH), jnp.int32),
        pltpu.VMEM((nch, CH), jnp.int32),
        pltpu.VMEM((CH, CP), jnp.float32),
        pltpu.VMEM((CH, CP), jnp.float32),
    ]
    round0_call = pl.kernel(
        functools.partial(_round0_body, nch),
        out_type=jax.ShapeDtypeStruct((2 * NP, CP), jnp.float32),
        mesh=_MESH,
        scratch_types=[
            pltpu.VMEM_SHARED((NP, CP), jnp.float32),
            pltpu.VMEM((RT, CP), jnp.float32),
        ] + edge_bufs + [
            pltpu.SemaphoreType.DMA,
            pltpu.SemaphoreType.DMA,
        ],
        compiler_params=_SC_PARAMS,
    )
    roundn_call = pl.kernel(
        functools.partial(_roundn_body, nch),
        out_type=jax.ShapeDtypeStruct((2 * NP, CP), jnp.float32),
        mesh=_MESH,
        scratch_types=[
            pltpu.VMEM_SHARED((NP, CP), jnp.float32),
            pltpu.VMEM_SHARED((NP, CP), jnp.float32),
        ] + edge_bufs + [
            pltpu.VMEM((UR, CP), jnp.float32),
            pltpu.VMEM((UR, CP), jnp.float32),
            pltpu.VMEM((UR, 2 * CP), jnp.float32),
            pltpu.SemaphoreType.DMA,
            pltpu.SemaphoreType.DMA,
        ],
        compiler_params=_SC_PARAMS,
    )

    accs = round0_call(hh0, src3, dst3)
    for _ in range(K - 1):
        accs = roundn_call(accs, qz, src3, dst3)

    final_call = pl.pallas_call(
        _final_body,
        grid=(NP // BLK,),
        in_specs=[
            pl.BlockSpec((BLK, CP), lambda i: (i, 0)),
            pl.BlockSpec((BLK, CP), lambda i: (i + NP // BLK, 0)),
            pl.BlockSpec((BLK, CP), lambda i: (i, 0)),
            pl.BlockSpec((BLK, 1), lambda i: (i, 0)),
        ],
        out_specs=pl.BlockSpec((BLK, 40), lambda i: (i, 0)),
        out_shape=jax.ShapeDtypeStruct((NP, 40), jnp.float32),
    )
    out = final_call(accs, accs, h0, dinv)
    return out[:n]


# speedup vs baseline: 1.2862x; 1.1281x over previous
import functools

import jax
import jax.numpy as jnp
from jax import lax
from jax.experimental import pallas as pl
from jax.experimental.pallas import tpu as pltpu
from jax.experimental.pallas import tpu_sc as plsc

NP = 10240
CP = 48
DW = 16
NW = 32
NS = 16
CH = 128
RT = NP // NS
UR = 80
BLK = 640
ALPHA = 0.1
K = 10

_MESH = plsc.VectorSubcoreMesh(core_axis_name="c", subcore_axis_name="s")
_SC_PARAMS = pltpu.CompilerParams(use_tc_tiling_on_sc=False)


def _zero_fill(ref, rows, width):
    @plsc.parallel_loop(0, rows, step=1, unroll=8)
    def _(i):
        for j in range(width // 16):
            ref[i, pl.ds(j * 16, 16)] = jnp.zeros((16,), jnp.float32)


def _ones_fill(ref, rows, width):
    @plsc.parallel_loop(0, rows, step=1, unroll=8)
    def _(i):
        for j in range(width // 16):
            ref[i, pl.ds(j * 16, 16)] = jnp.ones((16,), jnp.float32)


def _deg_body(nch, dst_hbm, degs_hbm, acc, zbuf, ones_v, dst_v):
    c = lax.axis_index("c")
    s = lax.axis_index("s")
    wid = c * NS + s
    _zero_fill(zbuf, RT, DW)
    pltpu.sync_copy(zbuf, acc.at[pl.ds(s * RT, RT)])
    plsc.subcore_barrier()
    _ones_fill(ones_v, CH, DW)
    pltpu.sync_copy(dst_hbm.at[wid], dst_v)

    def body(j, _):
        pltpu.sync_copy(ones_v, acc.at[dst_v.at[j]], add=True)
        return 0

    lax.fori_loop(0, nch, body, 0)
    plsc.subcore_barrier()
    pltpu.sync_copy(acc.at[pl.ds(s * RT, RT)],
                    degs_hbm.at[pl.ds(c * NP + s * RT, RT)])


def _edge_phase(nch, hhs, src_hbm, dst_hbm, accs_hbm,
                acc, src_v, dst_v, bufs, gsems, ssems, c, s):
    wid = c * NS + s
    pltpu.sync_copy(src_hbm.at[wid], src_v)
    pltpu.sync_copy(dst_hbm.at[wid], dst_v)
    plsc.subcore_barrier()

    def gather(j, b):
        pltpu.async_copy(hhs.at[src_v.at[j]], bufs[b], gsems[b])

    def gather_wait(b):
        pltpu.make_async_copy(hhs.at[src_v.at[0]], bufs[b], gsems[b]).wait()

    def scatter(j, b):
        pltpu.async_copy(bufs[b], acc.at[dst_v.at[j]], ssems[b], add=True)

    def scatter_wait(b):
        pltpu.make_async_copy(bufs[b], acc.at[dst_v.at[0]], ssems[b]).wait()

    gather(0, 0)
    gather(1, 1)
    gather(2, 2)
    gather_wait(0)
    scatter(0, 0)
    gather(3, 3)
    gather_wait(1)
    scatter(1, 1)

    def body(j4, _):
        for b in range(4):
            j = j4 * 4 + b
            scatter_wait(b)
            gather(j, b)
            bl = (b + 2) % 4
            gather_wait(bl)
            scatter(j - 2, bl)
        return 0

    lax.fori_loop(1, nch // 4, body, 0)
    gather_wait((nch - 2) % 4)
    scatter(nch - 2, (nch - 2) % 4)
    gather_wait((nch - 1) % 4)
    scatter(nch - 1, (nch - 1) % 4)
    for b in range(4):
        scatter_wait(b)
    plsc.subcore_barrier()
    pltpu.sync_copy(acc.at[pl.ds(s * RT, RT)],
                    accs_hbm.at[pl.ds(c * NP + s * RT, RT)])


def _round0_body(nch, hh_hbm, src_hbm, dst_hbm, accs_hbm,
                 acc, zbuf, src_v, dst_v, b0, b1, b2, b3,
                 g0, g1, g2, g3, s0, s1, s2, s3):
    c = lax.axis_index("c")
    s = lax.axis_index("s")

    @pl.when(c == 0)
    def _():
        pltpu.sync_copy(hh_hbm.at[pl.ds(s * RT, RT)], acc.at[pl.ds(s * RT, RT)])

    @pl.when(c != 0)
    def _():
        _zero_fill(zbuf, RT, CP)
        pltpu.sync_copy(zbuf, acc.at[pl.ds(s * RT, RT)])

    _edge_phase(nch, hh_hbm, src_hbm, dst_hbm, accs_hbm, acc, src_v, dst_v,
                (b0, b1, b2, b3), (g0, g1, g2, g3), (s0, s1, s2, s3), c, s)


def _roundn_body(nch, accp_hbm, qz_hbm, src_hbm, dst_hbm, accs_hbm,
                 hhs, acc, src_v, dst_v, b0, b1, b2, b3,
                 a0u, a1u, qzu, g0, g1, g2, g3, s0, s1, s2, s3):
    c = lax.axis_index("c")
    s = lax.axis_index("s")
    _upd = jax.named_scope("upd_phase")
    _upd.__enter__()
    for u in range(RT // UR):
        base = s * RT + u * UR
        pltpu.sync_copy(accp_hbm.at[pl.ds(base, UR)], a0u)
        pltpu.sync_copy(accp_hbm.at[pl.ds(NP + base, UR)], a1u)
        pltpu.sync_copy(qz_hbm.at[pl.ds(base, UR)], qzu)

        @plsc.parallel_loop(0, UR, step=1, unroll=8)
        def _(i):
            for j in range(CP // 16):
                sl = pl.ds(j * 16, 16)
                a0u[i, sl] = (qzu[i, sl] * (a0u[i, sl] + a1u[i, sl])
                              + qzu[i, pl.ds(CP + j * 16, 16)])
        pltpu.sync_copy(a0u, hhs.at[pl.ds(base, UR)])

        @pl.when(c == 0)
        def _():
            pltpu.sync_copy(a0u, acc.at[pl.ds(base, UR)])

    @pl.when(c != 0)
    def _():
        _zero_fill(a0u, UR, CP)
        for u in range(RT // UR):
            pltpu.sync_copy(a0u, acc.at[pl.ds(s * RT + u * UR, UR)])

    _upd.__exit__(None, None, None)
    with jax.named_scope("edge_phase"):
        _edge_phase(nch, hhs, src_hbm, dst_hbm, accs_hbm, acc, src_v, dst_v,
                    (b0, b1, b2, b3), (g0, g1, g2, g3), (s0, s1, s2, s3),
                    c, s)


def _prep_body(x_ref, w1_ref, b1_ref, w2_ref, b2_ref, dg0_ref, dg1_ref,
               h0_ref, hh0_ref, dinv_ref, qz_ref):
    i = pl.program_id(0)
    h = jnp.maximum(
        jnp.dot(x_ref[...], w1_ref[...], preferred_element_type=jnp.float32)
        + b1_ref[...], 0.0)
    h = jnp.dot(h, w2_ref[...], preferred_element_type=jnp.float32) + b2_ref[...]
    rows = i * BLK + lax.broadcasted_iota(jnp.int32, (BLK, 1), 0)
    h = jnp.where(rows < 10000, h, 0.0)
    deg = 1.0 + dg0_ref[:, 0:1] + dg1_ref[:, 0:1]
    dinv = lax.rsqrt(deg)
    h0_ref[...] = h
    hh0_ref[...] = h * dinv
    dinv_ref[...] = dinv
    qz_ref[:, :CP] = jnp.broadcast_to((1.0 - ALPHA) * dinv * dinv, (BLK, CP))
    qz_ref[:, CP:] = ALPHA * dinv * h


def _final_body(a0_ref, a1_ref, h0_ref, dinv_ref, out_ref):
    dinv = dinv_ref[...]
    hn = (1.0 - ALPHA) * dinv * (a0_ref[...] + a1_ref[...]) \
        + ALPHA * h0_ref[...]
    l = hn[:, :40]
    m = jnp.max(l, axis=1, keepdims=True)
    e = jnp.exp(l - m)
    out_ref[...] = l - m - jnp.log(jnp.sum(e, axis=1, keepdims=True))


def kernel(x, edge_index, W1, b1, W2, b2):
    n, d = x.shape
    e = edge_index.shape[1]
    hdim = W1.shape[1]
    c0 = W2.shape[1]
    ew = e // NW
    nch = -(-ew // CH)
    nch += nch % 2
    ewp = nch * CH

    x_pad = jnp.zeros((NP, d), x.dtype).at[:n].set(x)
    W2p = jnp.zeros((hdim, CP), W2.dtype).at[:, :c0].set(W2)
    b2p = jnp.zeros((CP,), b2.dtype).at[:c0].set(b2)

    pad = ewp - ew
    pad_s = ewp + 2 * CH - ew
    pad_src = 10000 + (jnp.arange(pad_s, dtype=jnp.int32) % 64)
    pad_dst = 10016 + (jnp.arange(pad, dtype=jnp.int32) % 128)
    src3 = jnp.concatenate(
        [edge_index[0].reshape(NW, ew),
         jnp.broadcast_to(pad_src, (NW, pad_s))], axis=1).reshape(NW, nch + 2, CH)
    dst3 = jnp.concatenate(
        [edge_index[1].reshape(NW, ew),
         jnp.broadcast_to(pad_dst, (NW, pad))], axis=1).reshape(NW, nch, CH)

    deg_call = pl.kernel(
        functools.partial(_deg_body, nch),
        out_type=jax.ShapeDtypeStruct((2 * NP, DW), jnp.float32),
        mesh=_MESH,
        scratch_types=[
            pltpu.VMEM_SHARED((NP, DW), jnp.float32),
            pltpu.VMEM((RT, DW), jnp.float32),
            pltpu.VMEM((CH, DW), jnp.float32),
            pltpu.VMEM((nch, CH), jnp.int32),
        ],
        compiler_params=_SC_PARAMS,
    )
    degs = deg_call(dst3)

    prep_call = pl.pallas_call(
        _prep_body,
        grid=(NP // BLK,),
        in_specs=[
            pl.BlockSpec((BLK, d), lambda i: (i, 0)),
            pl.BlockSpec((d, hdim), lambda i: (0, 0)),
            pl.BlockSpec((hdim,), lambda i: (0,)),
            pl.BlockSpec((hdim, CP), lambda i: (0, 0)),
            pl.BlockSpec((CP,), lambda i: (0,)),
            pl.BlockSpec((BLK, DW), lambda i: (i, 0)),
            pl.BlockSpec((BLK, DW), lambda i: (i + NP // BLK, 0)),
        ],
        out_specs=[
            pl.BlockSpec((BLK, CP), lambda i: (i, 0)),
            pl.BlockSpec((BLK, CP), lambda i: (i, 0)),
            pl.BlockSpec((BLK, 1), lambda i: (i, 0)),
            pl.BlockSpec((BLK, 2 * CP), lambda i: (i, 0)),
        ],
        out_shape=[
            jax.ShapeDtypeStruct((NP, CP), jnp.float32),
            jax.ShapeDtypeStruct((NP, CP), jnp.float32),
            jax.ShapeDtypeStruct((NP, 1), jnp.float32),
            jax.ShapeDtypeStruct((NP, 2 * CP), jnp.float32),
        ],
    )
    h0, hh0, dinv, qz = prep_call(x_pad, W1, b1, W2p, b2p, degs, degs)

    edge_bufs = [
        pltpu.VMEM((nch + 2, CH), jnp.int32),
        pltpu.VMEM((nch, CH), jnp.int32),
    ] + [pltpu.VMEM((CH, CP), jnp.float32)] * 4
    edge_sems = [pltpu.SemaphoreType.DMA] * 8
    round0_call = pl.kernel(
        functools.partial(_round0_body, nch),
        out_type=jax.ShapeDtypeStruct((2 * NP, CP), jnp.float32),
        mesh=_MESH,
        scratch_types=[
            pltpu.VMEM_SHARED((NP, CP), jnp.float32),
            pltpu.VMEM((RT, CP), jnp.float32),
        ] + edge_bufs + edge_sems,
        compiler_params=_SC_PARAMS,
    )
    roundn_call = pl.kernel(
        functools.partial(_roundn_body, nch),
        out_type=jax.ShapeDtypeStruct((2 * NP, CP), jnp.float32),
        mesh=_MESH,
        scratch_types=[
            pltpu.VMEM_SHARED((NP, CP), jnp.float32),
            pltpu.VMEM_SHARED((NP, CP), jnp.float32),
        ] + edge_bufs + [
            pltpu.VMEM((UR, CP), jnp.float32),
            pltpu.VMEM((UR, CP), jnp.float32),
            pltpu.VMEM((UR, 2 * CP), jnp.float32),
        ] + edge_sems,
        compiler_params=_SC_PARAMS,
    )

    accs = round0_call(hh0, src3, dst3)
    for _ in range(K - 1):
        accs = roundn_call(accs, qz, src3, dst3)

    final_call = pl.pallas_call(
        _final_body,
        grid=(NP // BLK,),
        in_specs=[
            pl.BlockSpec((BLK, CP), lambda i: (i, 0)),
            pl.BlockSpec((BLK, CP), lambda i: (i + NP // BLK, 0)),
            pl.BlockSpec((BLK, CP), lambda i: (i, 0)),
            pl.BlockSpec((BLK, 1), lambda i: (i, 0)),
        ],
        out_specs=pl.BlockSpec((BLK, 40), lambda i: (i, 0)),
        out_shape=jax.ShapeDtypeStruct((NP, 40), jnp.float32),
    )
    out = final_call(accs, accs, h0, dinv)
    return out[:n]


# speedup vs baseline: 1.4241x; 1.1073x over previous
import functools

import jax
import jax.numpy as jnp
from jax import lax
from jax.experimental import pallas as pl
from jax.experimental.pallas import tpu as pltpu
from jax.experimental.pallas import tpu_sc as plsc

NP = 10240
CP = 48
DW = 16
NW = 32
NS = 16
CH = 128
RT = NP // NS
UR = 80
BLK = 640
ALPHA = 0.1
K = 10

_MESH = plsc.VectorSubcoreMesh(core_axis_name="c", subcore_axis_name="s")
_SC_PARAMS = pltpu.CompilerParams(use_tc_tiling_on_sc=False)


def _zero_fill(ref, rows, width):
    @plsc.parallel_loop(0, rows, step=1, unroll=8)
    def _(i):
        for j in range(width // 16):
            ref[i, pl.ds(j * 16, 16)] = jnp.zeros((16,), jnp.float32)


def _ones_fill(ref, rows, width):
    @plsc.parallel_loop(0, rows, step=1, unroll=8)
    def _(i):
        for j in range(width // 16):
            ref[i, pl.ds(j * 16, 16)] = jnp.ones((16,), jnp.float32)


def _deg_body(nch, dst_hbm, degs_hbm, acc, zbuf, ones_v, dst_v):
    c = lax.axis_index("c")
    s = lax.axis_index("s")
    wid = c * NS + s
    _zero_fill(zbuf, RT, DW)
    pltpu.sync_copy(zbuf, acc.at[pl.ds(s * RT, RT)])
    plsc.subcore_barrier()
    _ones_fill(ones_v, CH, DW)
    pltpu.sync_copy(dst_hbm.at[wid], dst_v)

    def body(j, _):
        pltpu.sync_copy(ones_v, acc.at[dst_v.at[j]], add=True)
        return 0

    lax.fori_loop(0, nch, body, 0)
    plsc.subcore_barrier()
    pltpu.sync_copy(acc.at[pl.ds(s * RT, RT)],
                    degs_hbm.at[pl.ds(c * NP + s * RT, RT)])


def _edge_phase(nch, hhs, src_hbm, dst_hbm, accs_hbm,
                acc, src_v, dst_v, bufs, gsems, ssems, c, s):
    wid = c * NS + s
    pltpu.sync_copy(src_hbm.at[wid], src_v)
    pltpu.sync_copy(dst_hbm.at[wid], dst_v)
    plsc.subcore_barrier()

    def gather(j, b):
        pltpu.async_copy(hhs.at[src_v.at[j]], bufs[b], gsems[b])

    def gather_wait(b):
        pltpu.make_async_copy(hhs.at[src_v.at[0]], bufs[b], gsems[b]).wait()

    def scatter(j, b):
        pltpu.async_copy(bufs[b], acc.at[dst_v.at[j]], ssems[b], add=True)

    def scatter_wait(b):
        pltpu.make_async_copy(bufs[b], acc.at[dst_v.at[0]], ssems[b]).wait()

    gather(0, 0)
    gather(1, 1)
    gather(2, 2)
    gather_wait(0)
    scatter(0, 0)
    gather(3, 3)
    gather_wait(1)
    scatter(1, 1)

    def body(j4, _):
        for b in range(4):
            j = j4 * 4 + b
            scatter_wait(b)
            gather(j, b)
            bl = (b + 2) % 4
            gather_wait(bl)
            scatter(j - 2, bl)
        return 0

    lax.fori_loop(1, nch // 4, body, 0)
    gather_wait((nch - 2) % 4)
    scatter(nch - 2, (nch - 2) % 4)
    gather_wait((nch - 1) % 4)
    scatter(nch - 1, (nch - 1) % 4)
    for b in range(4):
        scatter_wait(b)
    plsc.subcore_barrier()
    pltpu.sync_copy(acc.at[pl.ds(s * RT, RT)],
                    accs_hbm.at[pl.ds(c * NP + s * RT, RT)])


def _round0_body(nch, hh_hbm, src_hbm, dst_hbm, accs_hbm,
                 acc, zbuf, src_v, dst_v, b0, b1, b2, b3,
                 g0, g1, g2, g3, s0, s1, s2, s3):
    c = lax.axis_index("c")
    s = lax.axis_index("s")

    @pl.when(c == 0)
    def _():
        pltpu.sync_copy(hh_hbm.at[pl.ds(s * RT, RT)], acc.at[pl.ds(s * RT, RT)])

    @pl.when(c != 0)
    def _():
        _zero_fill(zbuf, RT, CP)
        pltpu.sync_copy(zbuf, acc.at[pl.ds(s * RT, RT)])

    _edge_phase(nch, hh_hbm, src_hbm, dst_hbm, accs_hbm, acc, src_v, dst_v,
                (b0, b1, b2, b3), (g0, g1, g2, g3), (s0, s1, s2, s3), c, s)


def _roundn_body(nch, accp_hbm, qz_hbm, src_hbm, dst_hbm, accs_hbm,
                 hhs, acc, src_v, dst_v, b0, b1, b2, b3,
                 a0u, a1u, qzu, g0, g1, g2, g3, s0, s1, s2, s3):
    c = lax.axis_index("c")
    s = lax.axis_index("s")
    _upd = jax.named_scope("upd_phase")
    _upd.__enter__()
    for u in range(RT // UR):
        base = s * RT + u * UR
        pltpu.async_copy(accp_hbm.at[pl.ds(base, UR)], a0u, g0)
        pltpu.async_copy(accp_hbm.at[pl.ds(NP + base, UR)], a1u, g0)
        pltpu.async_copy(qz_hbm.at[pl.ds(base, UR)], qzu, g0)
        pltpu.make_async_copy(accp_hbm.at[pl.ds(base, UR)], a0u, g0).wait()
        pltpu.make_async_copy(accp_hbm.at[pl.ds(base, UR)], a1u, g0).wait()
        pltpu.make_async_copy(qz_hbm.at[pl.ds(base, UR)], qzu, g0).wait()

        @plsc.parallel_loop(0, UR, step=1, unroll=8)
        def _(i):
            for j in range(CP // 16):
                sl = pl.ds(j * 16, 16)
                a0u[i, sl] = (qzu[i, sl] * (a0u[i, sl] + a1u[i, sl])
                              + qzu[i, pl.ds(CP + j * 16, 16)])
        pltpu.sync_copy(a0u, hhs.at[pl.ds(base, UR)])

        @pl.when(c == 0)
        def _():
            pltpu.sync_copy(a0u, acc.at[pl.ds(base, UR)])

    @pl.when(c != 0)
    def _():
        _zero_fill(a0u, UR, CP)
        for u in range(RT // UR):
            pltpu.sync_copy(a0u, acc.at[pl.ds(s * RT + u * UR, UR)])

    _upd.__exit__(None, None, None)
    with jax.named_scope("edge_phase"):
        _edge_phase(nch, hhs, src_hbm, dst_hbm, accs_hbm, acc, src_v, dst_v,
                    (b0, b1, b2, b3), (g0, g1, g2, g3), (s0, s1, s2, s3),
                    c, s)


def _prep_body(x_ref, w1_ref, b1_ref, w2_ref, b2_ref, dg0_ref, dg1_ref,
               h0_ref, hh0_ref, dinv_ref, qz_ref):
    i = pl.program_id(0)
    h = jnp.maximum(
        jnp.dot(x_ref[...], w1_ref[...], preferred_element_type=jnp.float32)
        + b1_ref[...], 0.0)
    h = jnp.dot(h, w2_ref[...], preferred_element_type=jnp.float32) + b2_ref[...]
    rows = i * BLK + lax.broadcasted_iota(jnp.int32, (BLK, 1), 0)
    h = jnp.where(rows < 10000, h, 0.0)
    deg = 1.0 + dg0_ref[:, 0:1] + dg1_ref[:, 0:1]
    dinv = lax.rsqrt(deg)
    h0_ref[...] = h
    hh0_ref[...] = h * dinv
    dinv_ref[...] = dinv
    qz_ref[:, :CP] = jnp.broadcast_to((1.0 - ALPHA) * dinv * dinv, (BLK, CP))
    qz_ref[:, CP:] = ALPHA * dinv * h


def _final_body(a0_ref, a1_ref, h0_ref, dinv_ref, out_ref):
    dinv = dinv_ref[...]
    hn = (1.0 - ALPHA) * dinv * (a0_ref[...] + a1_ref[...]) \
        + ALPHA * h0_ref[...]
    l = hn[:, :40]
    m = jnp.max(l, axis=1, keepdims=True)
    e = jnp.exp(l - m)
    out_ref[...] = l - m - jnp.log(jnp.sum(e, axis=1, keepdims=True))


def kernel(x, edge_index, W1, b1, W2, b2):
    n, d = x.shape
    e = edge_index.shape[1]
    hdim = W1.shape[1]
    c0 = W2.shape[1]
    ew = e // NW
    nch = -(-ew // CH)
    nch += nch % 2
    ewp = nch * CH

    x_pad = jnp.zeros((NP, d), x.dtype).at[:n].set(x)
    W2p = jnp.zeros((hdim, CP), W2.dtype).at[:, :c0].set(W2)
    b2p = jnp.zeros((CP,), b2.dtype).at[:c0].set(b2)

    pad = ewp - ew
    pad_s = ewp + 2 * CH - ew
    pad_src = 10000 + (jnp.arange(pad_s, dtype=jnp.int32) % 64)
    pad_dst = 10016 + (jnp.arange(pad, dtype=jnp.int32) % 128)
    src3 = jnp.concatenate(
        [edge_index[0].reshape(NW, ew),
         jnp.broadcast_to(pad_src, (NW, pad_s))], axis=1).reshape(NW, nch + 2, CH)
    dst3 = jnp.concatenate(
        [edge_index[1].reshape(NW, ew),
         jnp.broadcast_to(pad_dst, (NW, pad))], axis=1).reshape(NW, nch, CH)

    deg_call = pl.kernel(
        functools.partial(_deg_body, nch),
        out_type=jax.ShapeDtypeStruct((2 * NP, DW), jnp.float32),
        mesh=_MESH,
        scratch_types=[
            pltpu.VMEM_SHARED((NP, DW), jnp.float32),
            pltpu.VMEM((RT, DW), jnp.float32),
            pltpu.VMEM((CH, DW), jnp.float32),
            pltpu.VMEM((nch, CH), jnp.int32),
        ],
        compiler_params=_SC_PARAMS,
    )
    degs = deg_call(dst3)

    prep_call = pl.pallas_call(
        _prep_body,
        grid=(NP // BLK,),
        in_specs=[
            pl.BlockSpec((BLK, d), lambda i: (i, 0)),
            pl.BlockSpec((d, hdim), lambda i: (0, 0)),
            pl.BlockSpec((hdim,), lambda i: (0,)),
            pl.BlockSpec((hdim, CP), lambda i: (0, 0)),
            pl.BlockSpec((CP,), lambda i: (0,)),
            pl.BlockSpec((BLK, DW), lambda i: (i, 0)),
            pl.BlockSpec((BLK, DW), lambda i: (i + NP // BLK, 0)),
        ],
        out_specs=[
            pl.BlockSpec((BLK, CP), lambda i: (i, 0)),
            pl.BlockSpec((BLK, CP), lambda i: (i, 0)),
            pl.BlockSpec((BLK, 1), lambda i: (i, 0)),
            pl.BlockSpec((BLK, 2 * CP), lambda i: (i, 0)),
        ],
        out_shape=[
            jax.ShapeDtypeStruct((NP, CP), jnp.float32),
            jax.ShapeDtypeStruct((NP, CP), jnp.float32),
            jax.ShapeDtypeStruct((NP, 1), jnp.float32),
            jax.ShapeDtypeStruct((NP, 2 * CP), jnp.float32),
        ],
    )
    h0, hh0, dinv, qz = prep_call(x_pad, W1, b1, W2p, b2p, degs, degs)

    edge_bufs = [
        pltpu.VMEM((nch + 2, CH), jnp.int32),
        pltpu.VMEM((nch, CH), jnp.int32),
    ] + [pltpu.VMEM((CH, CP), jnp.float32)] * 4
    edge_sems = [pltpu.SemaphoreType.DMA] * 8
    round0_call = pl.kernel(
        functools.partial(_round0_body, nch),
        out_type=jax.ShapeDtypeStruct((2 * NP, CP), jnp.float32),
        mesh=_MESH,
        scratch_types=[
            pltpu.VMEM_SHARED((NP, CP), jnp.float32),
            pltpu.VMEM((RT, CP), jnp.float32),
        ] + edge_bufs + edge_sems,
        compiler_params=_SC_PARAMS,
    )
    roundn_call = pl.kernel(
        functools.partial(_roundn_body, nch),
        out_type=jax.ShapeDtypeStruct((2 * NP, CP), jnp.float32),
        mesh=_MESH,
        scratch_types=[
            pltpu.VMEM_SHARED((NP, CP), jnp.float32),
            pltpu.VMEM_SHARED((NP, CP), jnp.float32),
        ] + edge_bufs + [
            pltpu.VMEM((UR, CP), jnp.float32),
            pltpu.VMEM((UR, CP), jnp.float32),
            pltpu.VMEM((UR, 2 * CP), jnp.float32),
        ] + edge_sems,
        compiler_params=_SC_PARAMS,
    )

    accs = round0_call(hh0, src3, dst3)
    for _ in range(K - 1):
        accs = roundn_call(accs, qz, src3, dst3)

    final_call = pl.pallas_call(
        _final_body,
        grid=(NP // BLK,),
        in_specs=[
            pl.BlockSpec((BLK, CP), lambda i: (i, 0)),
            pl.BlockSpec((BLK, CP), lambda i: (i + NP // BLK, 0)),
            pl.BlockSpec((BLK, CP), lambda i: (i, 0)),
            pl.BlockSpec((BLK, 1), lambda i: (i, 0)),
        ],
        out_specs=pl.BlockSpec((BLK, 40), lambda i: (i, 0)),
        out_shape=jax.ShapeDtypeStruct((NP, 40), jnp.float32),
    )
    out = final_call(accs, accs, h0, dinv)
    return out[:n]


# speedup vs baseline: 1.4317x; 1.0053x over previous
import functools

import jax
import jax.numpy as jnp
from jax import lax
from jax.experimental import pallas as pl
from jax.experimental.pallas import tpu as pltpu
from jax.experimental.pallas import tpu_sc as plsc

NP = 10240
CP = 48
DW = 16
NW = 32
NS = 16
CH = 128
RT = NP // NS
UR = 80
BLK = 640
ALPHA = 0.1
K = 10

_MESH = plsc.VectorSubcoreMesh(core_axis_name="c", subcore_axis_name="s")
_SC_PARAMS = pltpu.CompilerParams(use_tc_tiling_on_sc=False)


def _zero_fill(ref, rows, width):
    @plsc.parallel_loop(0, rows, step=1, unroll=8)
    def _(i):
        for j in range(width // 16):
            ref[i, pl.ds(j * 16, 16)] = jnp.zeros((16,), jnp.float32)


def _ones_fill(ref, rows, width):
    @plsc.parallel_loop(0, rows, step=1, unroll=8)
    def _(i):
        for j in range(width // 16):
            ref[i, pl.ds(j * 16, 16)] = jnp.ones((16,), jnp.float32)


def _deg_body(nch, dst_hbm, degs_hbm, acc, zbuf, ones_v, dst_v):
    c = lax.axis_index("c")
    s = lax.axis_index("s")
    wid = c * NS + s
    _zero_fill(zbuf, RT, DW)
    pltpu.sync_copy(zbuf, acc.at[pl.ds(s * RT, RT)])
    plsc.subcore_barrier()
    _ones_fill(ones_v, CH, DW)
    pltpu.sync_copy(dst_hbm.at[wid], dst_v)

    def body(j, _):
        pltpu.sync_copy(ones_v, acc.at[dst_v.at[j]], add=True)
        return 0

    lax.fori_loop(0, nch, body, 0)
    plsc.subcore_barrier()
    pltpu.sync_copy(acc.at[pl.ds(s * RT, RT)],
                    degs_hbm.at[pl.ds(c * NP + s * RT, RT)])


def _edge_phase(nch, hhs, src_hbm, dst_hbm, accs_hbm,
                acc, src_v, dst_v, bufs, gsems, ssems, c, s):
    wid = c * NS + s
    pltpu.sync_copy(src_hbm.at[wid], src_v)
    pltpu.sync_copy(dst_hbm.at[wid], dst_v)
    plsc.subcore_barrier()

    def gather(j, b):
        pltpu.async_copy(hhs.at[src_v.at[j]], bufs[b], gsems[b])

    def gather_wait(b):
        pltpu.make_async_copy(hhs.at[src_v.at[0]], bufs[b], gsems[b]).wait()

    def scatter(j, b):
        pltpu.async_copy(bufs[b], acc.at[dst_v.at[j]], ssems[b], add=True)

    def scatter_wait(b):
        pltpu.make_async_copy(bufs[b], acc.at[dst_v.at[0]], ssems[b]).wait()

    gather(0, 0)
    gather(1, 1)
    gather(2, 2)
    gather_wait(0)
    scatter(0, 0)
    gather(3, 3)
    gather_wait(1)
    scatter(1, 1)

    def body(j4, _):
        for b in range(4):
            j = j4 * 4 + b
            scatter_wait(b)
            gather(j, b)
            bl = (b + 2) % 4
            gather_wait(bl)
            scatter(j - 2, bl)
        return 0

    lax.fori_loop(1, nch // 4, body, 0)
    gather_wait((nch - 2) % 4)
    scatter(nch - 2, (nch - 2) % 4)
    gather_wait((nch - 1) % 4)
    scatter(nch - 1, (nch - 1) % 4)
    for b in range(4):
        scatter_wait(b)
    plsc.subcore_barrier()
    pltpu.sync_copy(acc.at[pl.ds(s * RT, RT)],
                    accs_hbm.at[pl.ds(c * NP + s * RT, RT)])


def _round0_body(nch, hh_hbm, src_hbm, dst_hbm, accs_hbm,
                 acc, zbuf, src_v, dst_v, b0, b1, b2, b3,
                 g0, g1, g2, g3, s0, s1, s2, s3):
    c = lax.axis_index("c")
    s = lax.axis_index("s")

    @pl.when(c == 0)
    def _():
        pltpu.sync_copy(hh_hbm.at[pl.ds(s * RT, RT)], acc.at[pl.ds(s * RT, RT)])

    @pl.when(c != 0)
    def _():
        _zero_fill(zbuf, RT, CP)
        pltpu.sync_copy(zbuf, acc.at[pl.ds(s * RT, RT)])

    _edge_phase(nch, hh_hbm, src_hbm, dst_hbm, accs_hbm, acc, src_v, dst_v,
                (b0, b1, b2, b3), (g0, g1, g2, g3), (s0, s1, s2, s3), c, s)


def _roundn_body(nch, accp_hbm, qz_hbm, src_hbm, dst_hbm, accs_hbm,
                 hhs, acc, src_v, dst_v, b0, b1, b2, b3,
                 a0u, a1u, qzu, g0, g1, g2, g3, s0, s1, s2, s3):
    c = lax.axis_index("c")
    s = lax.axis_index("s")
    _upd = jax.named_scope("upd_phase")
    _upd.__enter__()
    for u in range(RT // UR):
        base = s * RT + u * UR
        pltpu.async_copy(accp_hbm.at[pl.ds(base, UR)], a0u, g0)
        pltpu.async_copy(accp_hbm.at[pl.ds(NP + base, UR)], a1u, g0)
        pltpu.async_copy(qz_hbm.at[pl.ds(base, UR)], qzu, g0)
        pltpu.make_async_copy(accp_hbm.at[pl.ds(base, UR)], a0u, g0).wait()
        pltpu.make_async_copy(accp_hbm.at[pl.ds(base, UR)], a1u, g0).wait()
        pltpu.make_async_copy(qz_hbm.at[pl.ds(base, UR)], qzu, g0).wait()

        @plsc.parallel_loop(0, UR, step=1, unroll=8)
        def _(i):
            for j in range(CP // 16):
                sl = pl.ds(j * 16, 16)
                a0u[i, sl] = (qzu[i, sl] * (a0u[i, sl] + a1u[i, sl])
                              + qzu[i, pl.ds(CP + j * 16, 16)])
        pltpu.async_copy(a0u, hhs.at[pl.ds(base, UR)], g1)

        @pl.when(c == 0)
        def _():
            pltpu.async_copy(a0u, acc.at[pl.ds(base, UR)], g1)
            pltpu.make_async_copy(a0u, acc.at[pl.ds(base, UR)], g1).wait()

        pltpu.make_async_copy(a0u, hhs.at[pl.ds(base, UR)], g1).wait()

    @pl.when(c != 0)
    def _():
        _zero_fill(a0u, UR, CP)
        for u in range(RT // UR):
            pltpu.sync_copy(a0u, acc.at[pl.ds(s * RT + u * UR, UR)])

    _upd.__exit__(None, None, None)
    with jax.named_scope("edge_phase"):
        _edge_phase(nch, hhs, src_hbm, dst_hbm, accs_hbm, acc, src_v, dst_v,
                    (b0, b1, b2, b3), (g0, g1, g2, g3), (s0, s1, s2, s3),
                    c, s)


def _prep_body(x_ref, w1_ref, b1_ref, w2_ref, b2_ref, dg0_ref, dg1_ref,
               h0_ref, hh0_ref, dinv_ref, qz_ref):
    i = pl.program_id(0)
    h = jnp.maximum(
        jnp.dot(x_ref[...], w1_ref[...], preferred_element_type=jnp.float32)
        + b1_ref[...], 0.0)
    h = jnp.dot(h, w2_ref[...], preferred_element_type=jnp.float32) + b2_ref[...]
    rows = i * BLK + lax.broadcasted_iota(jnp.int32, (BLK, 1), 0)
    h = jnp.where(rows < 10000, h, 0.0)
    deg = 1.0 + dg0_ref[:, 0:1] + dg1_ref[:, 0:1]
    dinv = lax.rsqrt(deg)
    h0_ref[...] = h
    hh0_ref[...] = h * dinv
    dinv_ref[...] = dinv
    qz_ref[:, :CP] = jnp.broadcast_to((1.0 - ALPHA) * dinv * dinv, (BLK, CP))
    qz_ref[:, CP:] = ALPHA * dinv * h


def _final_body(a0_ref, a1_ref, h0_ref, dinv_ref, out_ref):
    dinv = dinv_ref[...]
    hn = (1.0 - ALPHA) * dinv * (a0_ref[...] + a1_ref[...]) \
        + ALPHA * h0_ref[...]
    l = hn[:, :40]
    m = jnp.max(l, axis=1, keepdims=True)
    e = jnp.exp(l - m)
    out_ref[...] = l - m - jnp.log(jnp.sum(e, axis=1, keepdims=True))


def kernel(x, edge_index, W1, b1, W2, b2):
    n, d = x.shape
    e = edge_index.shape[1]
    hdim = W1.shape[1]
    c0 = W2.shape[1]
    ew = e // NW
    nch = -(-ew // CH)
    nch += nch % 2
    ewp = nch * CH

    x_pad = jnp.zeros((NP, d), x.dtype).at[:n].set(x)
    W2p = jnp.zeros((hdim, CP), W2.dtype).at[:, :c0].set(W2)
    b2p = jnp.zeros((CP,), b2.dtype).at[:c0].set(b2)

    pad = ewp - ew
    pad_s = ewp + 2 * CH - ew
    pad_src = 10000 + (jnp.arange(pad_s, dtype=jnp.int32) % 64)
    pad_dst = 10016 + (jnp.arange(pad, dtype=jnp.int32) % 128)
    src3 = jnp.concatenate(
        [edge_index[0].reshape(NW, ew),
         jnp.broadcast_to(pad_src, (NW, pad_s))], axis=1).reshape(NW, nch + 2, CH)
    dst3 = jnp.concatenate(
        [edge_index[1].reshape(NW, ew),
         jnp.broadcast_to(pad_dst, (NW, pad))], axis=1).reshape(NW, nch, CH)

    deg_call = pl.kernel(
        functools.partial(_deg_body, nch),
        out_type=jax.ShapeDtypeStruct((2 * NP, DW), jnp.float32),
        mesh=_MESH,
        scratch_types=[
            pltpu.VMEM_SHARED((NP, DW), jnp.float32),
            pltpu.VMEM((RT, DW), jnp.float32),
            pltpu.VMEM((CH, DW), jnp.float32),
            pltpu.VMEM((nch, CH), jnp.int32),
        ],
        compiler_params=_SC_PARAMS,
    )
    degs = deg_call(dst3)

    prep_call = pl.pallas_call(
        _prep_body,
        grid=(NP // BLK,),
        in_specs=[
            pl.BlockSpec((BLK, d), lambda i: (i, 0)),
            pl.BlockSpec((d, hdim), lambda i: (0, 0)),
            pl.BlockSpec((hdim,), lambda i: (0,)),
            pl.BlockSpec((hdim, CP), lambda i: (0, 0)),
            pl.BlockSpec((CP,), lambda i: (0,)),
            pl.BlockSpec((BLK, DW), lambda i: (i, 0)),
            pl.BlockSpec((BLK, DW), lambda i: (i + NP // BLK, 0)),
        ],
        out_specs=[
            pl.BlockSpec((BLK, CP), lambda i: (i, 0)),
            pl.BlockSpec((BLK, CP), lambda i: (i, 0)),
            pl.BlockSpec((BLK, 1), lambda i: (i, 0)),
            pl.BlockSpec((BLK, 2 * CP), lambda i: (i, 0)),
        ],
        out_shape=[
            jax.ShapeDtypeStruct((NP, CP), jnp.float32),
            jax.ShapeDtypeStruct((NP, CP), jnp.float32),
            jax.ShapeDtypeStruct((NP, 1), jnp.float32),
            jax.ShapeDtypeStruct((NP, 2 * CP), jnp.float32),
        ],
    )
    h0, hh0, dinv, qz = prep_call(x_pad, W1, b1, W2p, b2p, degs, degs)

    edge_bufs = [
        pltpu.VMEM((nch + 2, CH), jnp.int32),
        pltpu.VMEM((nch, CH), jnp.int32),
    ] + [pltpu.VMEM((CH, CP), jnp.float32)] * 4
    edge_sems = [pltpu.SemaphoreType.DMA] * 8
    round0_call = pl.kernel(
        functools.partial(_round0_body, nch),
        out_type=jax.ShapeDtypeStruct((2 * NP, CP), jnp.float32),
        mesh=_MESH,
        scratch_types=[
            pltpu.VMEM_SHARED((NP, CP), jnp.float32),
            pltpu.VMEM((RT, CP), jnp.float32),
        ] + edge_bufs + edge_sems,
        compiler_params=_SC_PARAMS,
    )
    roundn_call = pl.kernel(
        functools.partial(_roundn_body, nch),
        out_type=jax.ShapeDtypeStruct((2 * NP, CP), jnp.float32),
        mesh=_MESH,
        scratch_types=[
            pltpu.VMEM_SHARED((NP, CP), jnp.float32),
            pltpu.VMEM_SHARED((NP, CP), jnp.float32),
        ] + edge_bufs + [
            pltpu.VMEM((UR, CP), jnp.float32),
            pltpu.VMEM((UR, CP), jnp.float32),
            pltpu.VMEM((UR, 2 * CP), jnp.float32),
        ] + edge_sems,
        compiler_params=_SC_PARAMS,
    )

    accs = round0_call(hh0, src3, dst3)
    for _ in range(K - 1):
        accs = roundn_call(accs, qz, src3, dst3)

    final_call = pl.pallas_call(
        _final_body,
        grid=(NP // BLK,),
        in_specs=[
            pl.BlockSpec((BLK, CP), lambda i: (i, 0)),
            pl.BlockSpec((BLK, CP), lambda i: (i + NP // BLK, 0)),
            pl.BlockSpec((BLK, CP), lambda i: (i, 0)),
            pl.BlockSpec((BLK, 1), lambda i: (i, 0)),
        ],
        out_specs=pl.BlockSpec((BLK, 40), lambda i: (i, 0)),
        out_shape=jax.ShapeDtypeStruct((NP, 40), jnp.float32),
    )
    out = final_call(accs, accs, h0, dinv)
    return out[:n]
